# Initial kernel scaffold; baseline (speedup 1.0000x reference)
import jax
import jax.numpy as jnp
from jax.experimental import pallas as pl


def kernel(x, edge_index, W2_0, W2_1, b2, W3_0, W3_1, W3_2, b3, W4_0, W4_1, W4_2, W4_3, b4, Watt, batt, Wfus, bfus, gamma, beta):
    raise NotImplementedError("write your pallas kernel here")


# speedup vs baseline: 7.2698x; 7.2698x over previous
import functools

import jax
import jax.numpy as jnp
from jax import lax
from jax.experimental import pallas as pl
from jax.experimental.pallas import tpu as pltpu
from jax.experimental.pallas import tpu_sc as plsc

_N = 10000
_E = 320000
_D = 128
_NC, _NS = 2, 16
_NW = _NC * _NS
_CH = 128
_EPAD = ((_E + _NW * _CH - 1) // (_NW * _CH)) * (_NW * _CH)
_CHW = _EPAD // (_NW * _CH)
_NPAD = 10240
_RPW = _NPAD // _NS

_mesh = plsc.VectorSubcoreMesh(
    core_axis_name="c", subcore_axis_name="s",
    num_cores=_NC, num_subcores=_NS)


def _deg_body(srcm, out, src_v, ones_v, zbuf, acc):
  c = lax.axis_index("c")
  s = lax.axis_index("s")
  wid = s * _NC + c
  r0 = s * _RPW
  for i in range(_RPW // 16):
    zbuf[pl.ds(i * 16, 16)] = jnp.zeros((16,), jnp.float32)
  pltpu.sync_copy(zbuf, acc.at[pl.ds(r0, _RPW)])
  for i in range(_CH // 16):
    ones_v[pl.ds(i * 16, 16)] = jnp.ones((16,), jnp.float32)
  pltpu.sync_copy(srcm.at[wid], src_v)
  plsc.subcore_barrier()

  def body(j, carry):
    pltpu.sync_copy(ones_v, acc.at[src_v.at[j]], add=True)
    return carry
  lax.fori_loop(0, _CHW, body, 0)

  plsc.subcore_barrier()
  pltpu.sync_copy(acc.at[pl.ds(r0, _RPW)], zbuf)
  pltpu.sync_copy(zbuf, out.at[pl.ds(c * _NPAD + r0, _RPW)])


def _degree(srcm):
  return pl.kernel(
      _deg_body,
      out_type=jax.ShapeDtypeStruct((_NC * _NPAD,), jnp.float32),
      mesh=_mesh,
      scratch_types=[
          pltpu.VMEM((_CHW, _CH), jnp.int32),
          pltpu.VMEM((_CH,), jnp.float32),
          pltpu.VMEM((_RPW,), jnp.float32),
          pltpu.VMEM_SHARED((_NPAD,), jnp.float32),
      ],
  )(srcm)


def _spass_body(g, srcm, dstm, z2, out, src_v, dst_v, rows_v, acc, sem):
  c = lax.axis_index("c")
  s = lax.axis_index("s")
  wid = s * _NC + c
  r0 = s * _RPW
  pltpu.sync_copy(z2.at[pl.ds(r0, _RPW)], acc.at[pl.ds(r0, _RPW)])
  pltpu.sync_copy(srcm.at[wid], src_v)
  pltpu.sync_copy(dstm.at[wid], dst_v)
  plsc.subcore_barrier()

  def body(j, carry):
    pltpu.async_copy(g.at[src_v.at[j]], rows_v, sem).wait()
    pltpu.sync_copy(rows_v, acc.at[dst_v.at[j]], add=True)
    return carry
  lax.fori_loop(0, _CHW, body, 0)

  plsc.subcore_barrier()
  pltpu.sync_copy(acc.at[pl.ds(r0, _RPW)], out.at[c, pl.ds(r0, _RPW)])


def _spass(g, srcm, dstm, z2):
  return pl.kernel(
      _spass_body,
      out_type=jax.ShapeDtypeStruct((_NC, _NPAD, _D), jnp.float32),
      mesh=_mesh,
      scratch_types=[
          pltpu.VMEM((_CHW, _CH), jnp.int32),
          pltpu.VMEM((_CHW, _CH), jnp.int32),
          pltpu.VMEM((_CH, _D), jnp.float32),
          pltpu.VMEM_SHARED((_NPAD, _D), jnp.float32),
          pltpu.SemaphoreType.DMA,
      ],
  )(g, srcm, dstm, z2)


def _glue_a_body(degp_ref, x_ref, dinv_ref, g0_ref):
  deg = degp_ref[0] + degp_ref[1]
  dinv = jnp.where(deg > 0.0, lax.rsqrt(deg), 0.0)
  dinv_ref[...] = dinv
  g0_ref[...] = x_ref[...] * dinv


def _glue_a(degp, x_pad):
  return pl.pallas_call(
      _glue_a_body,
      out_shape=[
          jax.ShapeDtypeStruct((_NPAD, 1), jnp.float32),
          jax.ShapeDtypeStruct((_NPAD, _D), jnp.float32),
      ],
  )(degp, x_pad)


def _glue_b_body(p_ref, dinv_ref, t1_ref, g1_ref):
  dinv = dinv_ref[...]
  t1 = -dinv * (p_ref[0] + p_ref[1])
  t1_ref[...] = t1
  g1_ref[...] = dinv * t1


def _glue_b(p, dinv):
  return pl.pallas_call(
      _glue_b_body,
      out_shape=[
          jax.ShapeDtypeStruct((_NPAD, _D), jnp.float32),
          jax.ShapeDtypeStruct((_NPAD, _D), jnp.float32),
      ],
  )(p, dinv)


def _glue_c_body(p_ref, dinv_ref, prev_ref, t_ref, g_ref):
  dinv = dinv_ref[...]
  t = -2.0 * dinv * (p_ref[0] + p_ref[1]) - prev_ref[...]
  t_ref[...] = t
  g_ref[...] = dinv * t


def _glue_c(p, dinv, prev):
  return pl.pallas_call(
      _glue_c_body,
      out_shape=[
          jax.ShapeDtypeStruct((_NPAD, _D), jnp.float32),
          jax.ShapeDtypeStruct((_NPAD, _D), jnp.float32),
      ],
  )(p, dinv, prev)


_BN = 2000


def _dense_body(t0_ref, t1_ref, t2_ref, t3_ref, wbig_ref, bcat_ref,
                watt_ref, batt_ref, wfus_ref, bfus_ref,
                h_ref, s1_ref, s2_ref):
  i = pl.program_id(0)
  tcat = jnp.concatenate(
      [t0_ref[...], t1_ref[...], t2_ref[...], t3_ref[...]], axis=1)
  feats = jnp.dot(tcat, wbig_ref[...],
                  preferred_element_type=jnp.float32,
                  precision=lax.Precision.HIGHEST) + bcat_ref[...]
  logits = jnp.dot(feats, watt_ref[...],
                   preferred_element_type=jnp.float32,
                   precision=lax.Precision.HIGHEST) + batt_ref[...]
  m = jnp.max(logits, axis=1, keepdims=True)
  e = jnp.exp(logits - m)
  sw = e / jnp.sum(e, axis=1, keepdims=True)
  fused = jnp.concatenate(
      [feats[:, k * _D:(k + 1) * _D] * sw[:, k:k + 1] for k in range(3)],
      axis=1)
  h = jnp.dot(fused, wfus_ref[...],
              preferred_element_type=jnp.float32,
              precision=lax.Precision.HIGHEST) + bfus_ref[...]
  h_ref[...] = h

  @pl.when(i == 0)
  def _():
    s1_ref[...] = jnp.zeros_like(s1_ref)
    s2_ref[...] = jnp.zeros_like(s2_ref)
  s1_ref[...] += jnp.sum(h, axis=0, keepdims=True)
  s2_ref[...] += jnp.sum(h * h, axis=0, keepdims=True)


def _dense(t0, t1, t2, t3, wbig, bcat, watt, batt, wfus, bfus):
  grid = _N // _BN
  row_spec = pl.BlockSpec((_BN, _D), lambda i: (i, 0))
  full = lambda s: pl.BlockSpec(s, lambda i: tuple(0 for _ in s))
  return pl.pallas_call(
      _dense_body,
      grid=(grid,),
      in_specs=[
          row_spec, row_spec, row_spec, row_spec,
          full((4 * _D, 3 * _D)), full((1, 3 * _D)),
          full((3 * _D, 3)), full((1, 3)),
          full((3 * _D, _D)), full((1, _D)),
      ],
      out_specs=[
          pl.BlockSpec((_BN, _D), lambda i: (i, 0)),
          full((1, _D)), full((1, _D)),
      ],
      out_shape=[
          jax.ShapeDtypeStruct((_N, _D), jnp.float32),
          jax.ShapeDtypeStruct((1, _D), jnp.float32),
          jax.ShapeDtypeStruct((1, _D), jnp.float32),
      ],
  )(t0, t1, t2, t3, wbig, bcat, watt, batt, wfus, bfus)


def _norm_body(h_ref, s1_ref, s2_ref, gamma_ref, beta_ref, o_ref):
  mean = s1_ref[...] / _N
  var = s2_ref[...] / _N - mean * mean
  scale = lax.rsqrt(var + 1e-5) * gamma_ref[...]
  o_ref[...] = jnp.maximum((h_ref[...] - mean) * scale + beta_ref[...], 0.0)


def _norm(h, s1, s2, gamma, beta):
  row_spec = pl.BlockSpec((_BN, _D), lambda i: (i, 0))
  full = lambda s: pl.BlockSpec(s, lambda i: tuple(0 for _ in s))
  return pl.pallas_call(
      _norm_body,
      grid=(_N // _BN,),
      in_specs=[row_spec, full((1, _D)), full((1, _D)),
                full((1, _D)), full((1, _D))],
      out_specs=row_spec,
      out_shape=jax.ShapeDtypeStruct((_N, _D), jnp.float32),
  )(h, s1, s2, gamma, beta)


def kernel(x, edge_index, W2_0, W2_1, b2, W3_0, W3_1, W3_2, b3,
           W4_0, W4_1, W4_2, W4_3, b4, Watt, batt, Wfus, bfus, gamma, beta):
  f32 = jnp.float32

  pad = _EPAD - _E
  src = jnp.concatenate(
      [edge_index[0], jnp.full((pad,), _NPAD - 1, jnp.int32)])
  dst = jnp.concatenate(
      [edge_index[1], jnp.full((pad,), _NPAD - 1, jnp.int32)])
  srcm = src.reshape(_NW, _CHW, _CH)
  dstm = dst.reshape(_NW, _CHW, _CH)
  x_pad = jnp.zeros((_NPAD, _D), f32).at[:_N].set(x)
  z2 = jnp.zeros((_NPAD, _D), f32)

  degp = _degree(srcm)
  dinv, g0 = _glue_a(degp.reshape(_NC, _NPAD, 1), x_pad)

  p0 = _spass(g0, srcm, dstm, z2)
  t1, g1 = _glue_b(p0, dinv)
  p1 = _spass(g1, srcm, dstm, z2)
  t2, g2 = _glue_c(p1, dinv, x_pad)
  p2 = _spass(g2, srcm, dstm, z2)
  t3, _ = _glue_c(p2, dinv, t1)

  zDD = jnp.zeros((_D, _D), f32)
  wbig = jnp.concatenate([
      jnp.concatenate([W2_0, W3_0, W4_0], axis=1),
      jnp.concatenate([W2_1, W3_1, W4_1], axis=1),
      jnp.concatenate([zDD, W3_2, W4_2], axis=1),
      jnp.concatenate([zDD, zDD, W4_3], axis=1),
  ], axis=0)
  bcat = jnp.concatenate([b2, b3, b4]).reshape(1, 3 * _D)
  h, s1, s2 = _dense(x_pad[:_N], t1[:_N], t2[:_N], t3[:_N],
                     wbig, bcat, Watt, batt.reshape(1, 3),
                     Wfus, bfus.reshape(1, _D))
  return _norm(h, s1, s2, gamma.reshape(1, _D), beta.reshape(1, _D))
